# baseline (speedup 1.0000x reference)
import functools

import jax
import jax.numpy as jnp
from jax import lax
from jax.experimental import pallas as pl
from jax.experimental.pallas import tpu as pltpu
from jax.experimental.pallas import tpu_sc as plsc

_N = 65536
_DIM = 64
_C = 128
_K = 32
_KB = 8
_EPS = 1e-6
_MARGIN = 1.0
_B = 256
_NB = _N // _B
_NEG = -1e30
_CHUNK = 256

_NW = 32
_ROWS_PER_TILE = (_NB * _KB) // _NW


def _lanes():
  return lax.broadcasted_iota(jnp.int32, (1, _C), 1)


def _tc_a_body(x_ref, y2_ref, y0_ref,
               cp_ref, cn_ref, mp_out, mn_out, ut_out, un_out,
               csum_ref, cnt_ref, ut_ref, un_ref, mp_ref, mn_ref):
  p = pl.program_id(0)
  b = pl.program_id(1)
  lanes = _lanes()

  @pl.when(jnp.logical_and(p == 0, b == 0))
  def _init():
    csum_ref[...] = jnp.zeros((_DIM, _C), jnp.float32)
    cnt_ref[...] = jnp.zeros((1, _C), jnp.float32)

  @pl.when(p == 0)
  def _accum():
    oh = (y2_ref[...] == lanes).astype(jnp.float32)
    csum_ref[...] += lax.dot_general(
        x_ref[...], oh, (((0,), (0,)), ((), ())),
        preferred_element_type=jnp.float32)
    cnt_ref[...] += jnp.sum(oh, axis=0, keepdims=True)

  @pl.when(jnp.logical_and(p == 1, b == 0))
  def _centers():
    ut = csum_ref[...] / cnt_ref[...] + _EPS
    ut_ref[...] = ut
    un = jnp.sum(ut * ut, axis=0, keepdims=True)
    un_ref[...] = un
    ut_out[...] = ut
    un_out[...] = un

  @pl.when(p == 1)
  def _distance_block():
    x_blk = x_ref[...]
    cross = lax.dot_general(
        x_blk, ut_ref[...], (((1,), (0,)), ((), ())),
        preferred_element_type=jnp.float32)
    xnorm = jnp.sum(x_blk * x_blk, axis=1, keepdims=True)
    d2 = jnp.maximum(un_ref[...] - 2.0 * cross + xnorm, 0.0)
    ppos = jnp.where(y2_ref[...] == lanes, d2, _NEG)
    pneg = jnp.where(y0_ref[...] != lanes, -d2, _NEG)

    def extract(dst_ref, pv):
      def body(k, carry):
        pv, _ = carry
        m = jnp.max(pv, axis=0, keepdims=True)
        dst_ref[pl.ds(k, 1), :] = m
        return jnp.where(pv == m, _NEG, pv), m
      _, last = lax.fori_loop(0, _KB, body,
                              (pv, jnp.full((1, _C), _NEG, jnp.float32)))
      return last

    mp_ref[pl.ds(b, 1), :] = extract(cp_ref, ppos)
    mn_ref[pl.ds(b, 1), :] = extract(cn_ref, pneg)

  @pl.when(jnp.logical_and(p == 1, b == _NB - 1))
  def _emit_m():
    mp_out[...] = mp_ref[...]
    mn_out[...] = mn_ref[...]


def _tc_a(x, y2, y0):
  return pl.pallas_call(
      _tc_a_body,
      grid=(2, _NB),
      in_specs=[
          pl.BlockSpec((_B, _DIM), lambda p, b: (b, 0)),
          pl.BlockSpec((_B, 1), lambda p, b: (b, 0)),
          pl.BlockSpec((_B, 1), lambda p, b: (b, 0)),
      ],
      out_specs=[
          pl.BlockSpec((_KB, _C), lambda p, b: (b, 0)),
          pl.BlockSpec((_KB, _C), lambda p, b: (b, 0)),
          pl.BlockSpec((_NB, _C), lambda p, b: (0, 0)),
          pl.BlockSpec((_NB, _C), lambda p, b: (0, 0)),
          pl.BlockSpec((_DIM, _C), lambda p, b: (0, 0)),
          pl.BlockSpec((1, _C), lambda p, b: (0, 0)),
      ],
      out_shape=[
          jax.ShapeDtypeStruct((_NB * _KB, _C), jnp.float32),
          jax.ShapeDtypeStruct((_NB * _KB, _C), jnp.float32),
          jax.ShapeDtypeStruct((_NB, _C), jnp.float32),
          jax.ShapeDtypeStruct((_NB, _C), jnp.float32),
          jax.ShapeDtypeStruct((_DIM, _C), jnp.float32),
          jax.ShapeDtypeStruct((1, _C), jnp.float32),
      ],
      scratch_shapes=[
          pltpu.VMEM((_DIM, _C), jnp.float32),
          pltpu.VMEM((1, _C), jnp.float32),
          pltpu.VMEM((_DIM, _C), jnp.float32),
          pltpu.VMEM((1, _C), jnp.float32),
          pltpu.VMEM((_NB, _C), jnp.float32),
          pltpu.VMEM((_NB, _C), jnp.float32),
      ],
      compiler_params=pltpu.CompilerParams(
          dimension_semantics=("arbitrary", "arbitrary")),
  )(x, y2, y0)


def _s16(v):
  k, _ = plsc.sort_key_val(v, v, descending=True)
  return k


def _merge16(a, b):
  br = lax.rev(b, (0,))
  return _s16(jnp.maximum(a, br)), _s16(jnp.minimum(a, br))


def _merge32(ahi, alo, bhi, blo):
  h1 = jnp.maximum(ahi, lax.rev(blo, (0,)))
  h2 = jnp.maximum(alo, lax.rev(bhi, (0,)))
  return _s16(jnp.maximum(h1, h2)), _s16(jnp.minimum(h1, h2))


_SC_MESH = plsc.VectorSubcoreMesh(core_axis_name="c", subcore_axis_name="s")


def _sc_b1(cp, cn):
  @functools.partial(
      pl.kernel, mesh=_SC_MESH,
      out_type=[
          jax.ShapeDtypeStruct((_NW, _C, _K), jnp.float32),
          jax.ShapeDtypeStruct((_NW, _C, _K), jnp.float32),
      ],
      compiler_params=pltpu.CompilerParams(needs_layout_passes=False),
      scratch_types=[
          pltpu.VMEM((_ROWS_PER_TILE, _C), jnp.float32),
          pltpu.VMEM((_C, _K), jnp.float32),
      ],
  )
  def b1(cp_hbm, cn_hbm, pp_out, pn_out, buf, ob):
    cid = lax.axis_index("c")
    sid = lax.axis_index("s")
    wid = cid * 16 + sid
    rows = [lax.iota(jnp.int32, 16) + 16 * g
            for g in range(_ROWS_PER_TILE // 16)]

    def one_side(src_hbm, dst_hbm):
      pltpu.sync_copy(src_hbm.at[pl.ds(wid * _ROWS_PER_TILE, _ROWS_PER_TILE), :],
                      buf)

      def cls(c, _):
        cols = jnp.full((16,), 0, jnp.int32) + c
        v = [_s16(plsc.load_gather(buf, [rows[g], cols]))
             for g in range(4)]
        ahi, alo = _merge16(v[0], v[1])
        bhi, blo = _merge16(v[2], v[3])
        hi, lo = _merge32(ahi, alo, bhi, blo)
        ob[c, pl.ds(0, 16)] = hi
        ob[c, pl.ds(16, 16)] = lo
        return 0
      lax.fori_loop(0, _C, cls, 0)
      pltpu.sync_copy(ob, dst_hbm.at[wid])

    one_side(cp_hbm, pp_out)
    one_side(cn_hbm, pn_out)

  return b1(cp, cn)


def _sc_b2(pp, pn, mp, mn):
  cpt = _C // _NW

  @functools.partial(
      pl.kernel, mesh=_SC_MESH,
      out_type=[
          jax.ShapeDtypeStruct((_C, _K), jnp.float32),
          jax.ShapeDtypeStruct((_C, _K), jnp.float32),
          jax.ShapeDtypeStruct((_NW, 16), jnp.int32),
      ],
      compiler_params=pltpu.CompilerParams(needs_layout_passes=False),
      scratch_types=[
          pltpu.VMEM((_NW, _K), jnp.float32),
          pltpu.VMEM((_NB, _C), jnp.float32),
          pltpu.VMEM((cpt, _K), jnp.float32),
          pltpu.VMEM((16,), jnp.float32),
          pltpu.VMEM((16,), jnp.int32),
          pltpu.SemaphoreType.DMA,
      ],
  )
  def b2(pp_hbm, pn_hbm, mp_hbm, mn_hbm, dp_out, dn_out, tg_out,
         lists, mbuf, obuf, tbuf, trow, sem):
    cid = lax.axis_index("c")
    sid = lax.axis_index("s")
    wid = cid * 16 + sid
    c0 = wid * cpt
    rows16 = [lax.iota(jnp.int32, 16) + 16 * g for g in range(_NB // 16)]

    def side(src_hbm, m_hbm, dst_hbm, trig_in):
      pltpu.sync_copy(m_hbm, mbuf)

      def cls(j, trig):
        c = c0 + j
        for t in range(_NW):
          pltpu.async_copy(src_hbm.at[t, c], lists.at[t], sem)
        for t in range(_NW):
          pltpu.make_async_copy(src_hbm.at[t, c], lists.at[t], sem).wait()
        hi = lists[0, pl.ds(0, 16)]
        lo = lists[0, pl.ds(16, 16)]

        def mrg(t, carry):
          hi, lo = carry
          return _merge32(hi, lo, lists[t, pl.ds(0, 16)],
                          lists[t, pl.ds(16, 16)])
        hi, lo = lax.fori_loop(1, _NW, mrg, (hi, lo))
        obuf[j, pl.ds(0, 16)] = hi
        obuf[j, pl.ds(16, 16)] = lo
        tbuf[...] = lo
        t32 = plsc.load_gather(tbuf, [jnp.full((16,), 15, jnp.int32)])
        cols = jnp.full((16,), 0, jnp.int32) + c
        acc = jnp.full((16,), _NEG, jnp.float32)
        for g in range(_NB // 16):
          acc = jnp.maximum(acc, plsc.load_gather(mbuf, [rows16[g], cols]))
        hit = (acc >= t32) & (t32 > -1e29)
        nhit = lax.reduce_max(hit.astype(jnp.int32), (0,))
        live = (c < 100).astype(jnp.int32)
        return trig | (nhit * live)
      trig = lax.fori_loop(0, cpt, cls, trig_in)
      pltpu.sync_copy(obuf, dst_hbm.at[pl.ds(c0, cpt), :])
      return trig

    trig = side(pp_hbm, mp_hbm, dp_out, 0)
    trig = side(pn_hbm, mn_hbm, dn_out, trig)
    trow[...] = jnp.full((16,), 0, jnp.int32) + trig
    pltpu.sync_copy(trow, tg_out.at[wid])

  return b2(pp, pn, mp, mn)


def _tc_c_body(x_ref, y2_ref, y0_ref, dpt_ref, dnt_ref, tg_ref,
               ut_in, un_in, out_ref,
               bp_ref, bn_ref, dp_ref, dn_ref, trig_ref):
  b = pl.program_id(0)
  lanes = _lanes()

  @pl.when(b == 0)
  def _check():
    trig_ref[0] = (jnp.sum(tg_ref[...]) > 0).astype(jnp.int32)
    d_ap = jnp.sqrt(jnp.maximum(dpt_ref[...], 0.0))
    d_an = jnp.sqrt(jnp.maximum(-dnt_ref[...], 0.0))
    hinge = jnp.maximum(d_ap - d_an + _MARGIN, 0.0)
    lo = jnp.sum(hinge, axis=1, keepdims=True) / float(_K)
    present = dpt_ref[...][:, 0:1] > -1e29
    lo = jnp.where(present, lo, 0.0)
    n_present = jnp.sum(present.astype(jnp.float32))
    out_ref[...] = jnp.sum(lo, axis=0, keepdims=True) / n_present

  @pl.when(trig_ref[0] != 0)
  def _slow_block():
    x_blk = x_ref[...]
    cross = lax.dot_general(
        x_blk, ut_in[...], (((1,), (0,)), ((), ())),
        preferred_element_type=jnp.float32)
    xnorm = jnp.sum(x_blk * x_blk, axis=1, keepdims=True)
    d2 = jnp.maximum(un_in[...] - 2.0 * cross + xnorm, 0.0)
    ppos = jnp.where(y2_ref[...] == lanes, d2, _NEG)
    pneg = jnp.where(y0_ref[...] != lanes, -d2, _NEG)

    def extract(dst_ref, base, pv):
      def body(k, carry):
        pv, _ = carry
        m = jnp.max(pv, axis=0, keepdims=True)
        dst_ref[pl.ds(base + k, 1), :] = m
        return jnp.where(pv == m, _NEG, pv), m
      lax.fori_loop(0, _K, body,
                    (pv, jnp.full((1, _C), _NEG, jnp.float32)))
    extract(bp_ref, b * _K, ppos)
    extract(bn_ref, b * _K, pneg)

  @pl.when(jnp.logical_and(b == _NB - 1, trig_ref[0] != 0))
  def _slow_finish():
    def merge(src_ref, dst_ref):
      nchunks = (_NB * _K) // _CHUNK
      def kbody(k, _):
        def cmax(ci, m):
          blk = src_ref[pl.ds(ci * _CHUNK, _CHUNK), :]
          return jnp.maximum(m, jnp.max(blk, axis=0, keepdims=True))
        m = lax.fori_loop(0, nchunks, cmax,
                          jnp.full((1, _C), _NEG, jnp.float32))
        dst_ref[pl.ds(k, 1), :] = m
        def cupd(ci, _):
          blk = src_ref[pl.ds(ci * _CHUNK, _CHUNK), :]
          src_ref[pl.ds(ci * _CHUNK, _CHUNK), :] = jnp.where(
              blk == m, _NEG, blk)
          return 0
        lax.fori_loop(0, nchunks, cupd, 0)
        return 0
      lax.fori_loop(0, _K, kbody, 0)
    merge(bp_ref, dp_ref)
    merge(bn_ref, dn_ref)
    d_ap = jnp.sqrt(jnp.maximum(dp_ref[...], 0.0))
    d_an = jnp.sqrt(jnp.maximum(-dn_ref[...], 0.0))
    hinge = jnp.maximum(d_ap - d_an + _MARGIN, 0.0)
    lo = jnp.sum(hinge, axis=0, keepdims=True) / float(_K)
    present = dp_ref[...][0:1, :] > -1e29
    lo = jnp.where(present, lo, 0.0)
    n_present = jnp.sum(present.astype(jnp.float32))
    out_ref[...] = (jnp.sum(lo, axis=1, keepdims=True) / n_present).reshape(1, 1)


def _tc_c(x, y2, y0, dpt, dnt, tg, ut, un):
  return pl.pallas_call(
      _tc_c_body,
      grid=(_NB,),
      in_specs=[
          pl.BlockSpec((_B, _DIM), lambda b: (b, 0)),
          pl.BlockSpec((_B, 1), lambda b: (b, 0)),
          pl.BlockSpec((_B, 1), lambda b: (b, 0)),
          pl.BlockSpec((_C, _K), lambda b: (0, 0)),
          pl.BlockSpec((_C, _K), lambda b: (0, 0)),
          pl.BlockSpec((_NW, 16), lambda b: (0, 0)),
          pl.BlockSpec((_DIM, _C), lambda b: (0, 0)),
          pl.BlockSpec((1, _C), lambda b: (0, 0)),
      ],
      out_specs=pl.BlockSpec((1, 1), lambda b: (0, 0)),
      out_shape=jax.ShapeDtypeStruct((1, 1), jnp.float32),
      scratch_shapes=[
          pltpu.VMEM((_NB * _K, _C), jnp.float32),
          pltpu.VMEM((_NB * _K, _C), jnp.float32),
          pltpu.VMEM((_K, _C), jnp.float32),
          pltpu.VMEM((_K, _C), jnp.float32),
          pltpu.SMEM((1,), jnp.int32),
      ],
      compiler_params=pltpu.CompilerParams(
          dimension_semantics=("arbitrary",)),
  )(x, y2, y0, dpt, dnt, tg, ut, un)


@functools.partial(jax.jit, static_argnames=())
def kernel(x, y):
  y2 = y[:, 2:3]
  y0 = y[:, 0:1]
  cp, cn, mp, mn, ut, un = _tc_a(x, y2, y0)
  pp, pn = _sc_b1(cp, cn)
  dpt, dnt, tg = _sc_b2(pp, pn, mp, mn)
  out = _tc_c(x, y2, y0, dpt, dnt, tg, ut, un)
  return out.reshape((1,))


# speedup vs baseline: 1.0296x; 1.0296x over previous
import functools

import jax
import jax.numpy as jnp
from jax import lax
from jax.experimental import pallas as pl
from jax.experimental.pallas import tpu as pltpu
from jax.experimental.pallas import tpu_sc as plsc

_N = 65536
_DIM = 64
_C = 128
_K = 32
_KB = 6
_EPS = 1e-6
_MARGIN = 1.0
_B = 256
_NB = _N // _B
_NEG = -1e30
_CHUNK = 256

_NW = 32
_ROWS_PER_TILE = (_NB * _KB) // _NW


def _lanes():
  return lax.broadcasted_iota(jnp.int32, (1, _C), 1)


def _tc_a_body(x_ref, y2_ref, y0_ref,
               cp_out, cn_out, mp_out, mn_out, ut_out, un_out,
               csum_ref, cnt_ref, ut_ref, un_ref, mp_ref, mn_ref,
               cp_ref, cn_ref):
  p = pl.program_id(0)
  b = pl.program_id(1)
  lanes = _lanes()

  @pl.when(jnp.logical_and(p == 0, b == 0))
  def _init():
    csum_ref[...] = jnp.zeros((_DIM, _C), jnp.float32)
    cnt_ref[...] = jnp.zeros((1, _C), jnp.float32)

  @pl.when(p == 0)
  def _accum():
    oh = (y2_ref[...] == lanes).astype(jnp.float32)
    csum_ref[...] += lax.dot_general(
        x_ref[...], oh, (((0,), (0,)), ((), ())),
        preferred_element_type=jnp.float32)
    cnt_ref[...] += jnp.sum(oh, axis=0, keepdims=True)

  @pl.when(jnp.logical_and(p == 1, b == 0))
  def _centers():
    ut = csum_ref[...] / cnt_ref[...] + _EPS
    ut_ref[...] = ut
    un = jnp.sum(ut * ut, axis=0, keepdims=True)
    un_ref[...] = un
    ut_out[...] = ut
    un_out[...] = un

  @pl.when(p == 1)
  def _distance_block():
    x_blk = x_ref[...]
    cross = lax.dot_general(
        x_blk, ut_ref[...], (((1,), (0,)), ((), ())),
        preferred_element_type=jnp.float32)
    xnorm = jnp.sum(x_blk * x_blk, axis=1, keepdims=True)
    d2 = jnp.maximum(un_ref[...] - 2.0 * cross + xnorm, 0.0)
    ppos = jnp.where(y2_ref[...] == lanes, d2, _NEG)
    pneg = jnp.where(y0_ref[...] != lanes, -d2, _NEG)

    def extract(dst_ref, pv):
      def body(k, carry):
        pv, _ = carry
        m = jnp.max(pv, axis=0, keepdims=True)
        dst_ref[pl.ds(b * _KB + k, 1), :] = m
        return jnp.where(pv == m, _NEG, pv), m
      _, last = lax.fori_loop(0, _KB, body,
                              (pv, jnp.full((1, _C), _NEG, jnp.float32)))
      return last

    mp_ref[pl.ds(b, 1), :] = extract(cp_ref, ppos)
    mn_ref[pl.ds(b, 1), :] = extract(cn_ref, pneg)

  @pl.when(jnp.logical_and(p == 1, b == _NB - 1))
  def _emit_m():
    mp_out[...] = mp_ref[...]
    mn_out[...] = mn_ref[...]
    cp_out[...] = cp_ref[...]
    cn_out[...] = cn_ref[...]


def _tc_a(x, y2, y0):
  return pl.pallas_call(
      _tc_a_body,
      grid=(2, _NB),
      in_specs=[
          pl.BlockSpec((_B, _DIM), lambda p, b: (b, 0)),
          pl.BlockSpec((_B, 1), lambda p, b: (b, 0)),
          pl.BlockSpec((_B, 1), lambda p, b: (b, 0)),
      ],
      out_specs=[
          pl.BlockSpec((_NB * _KB, _C), lambda p, b: (0, 0)),
          pl.BlockSpec((_NB * _KB, _C), lambda p, b: (0, 0)),
          pl.BlockSpec((_NB, _C), lambda p, b: (0, 0)),
          pl.BlockSpec((_NB, _C), lambda p, b: (0, 0)),
          pl.BlockSpec((_DIM, _C), lambda p, b: (0, 0)),
          pl.BlockSpec((1, _C), lambda p, b: (0, 0)),
      ],
      out_shape=[
          jax.ShapeDtypeStruct((_NB * _KB, _C), jnp.float32),
          jax.ShapeDtypeStruct((_NB * _KB, _C), jnp.float32),
          jax.ShapeDtypeStruct((_NB, _C), jnp.float32),
          jax.ShapeDtypeStruct((_NB, _C), jnp.float32),
          jax.ShapeDtypeStruct((_DIM, _C), jnp.float32),
          jax.ShapeDtypeStruct((1, _C), jnp.float32),
      ],
      scratch_shapes=[
          pltpu.VMEM((_DIM, _C), jnp.float32),
          pltpu.VMEM((1, _C), jnp.float32),
          pltpu.VMEM((_DIM, _C), jnp.float32),
          pltpu.VMEM((1, _C), jnp.float32),
          pltpu.VMEM((_NB, _C), jnp.float32),
          pltpu.VMEM((_NB, _C), jnp.float32),
          pltpu.VMEM((_NB * _KB, _C), jnp.float32),
          pltpu.VMEM((_NB * _KB, _C), jnp.float32),
      ],
      compiler_params=pltpu.CompilerParams(
          dimension_semantics=("arbitrary", "arbitrary")),
  )(x, y2, y0)


def _s16(v):
  k, _ = plsc.sort_key_val(v, v, descending=True)
  return k


def _merge16(a, b):
  br = lax.rev(b, (0,))
  return _s16(jnp.maximum(a, br)), _s16(jnp.minimum(a, br))


def _merge32(ahi, alo, bhi, blo):
  h1 = jnp.maximum(ahi, lax.rev(blo, (0,)))
  h2 = jnp.maximum(alo, lax.rev(bhi, (0,)))
  return _s16(jnp.maximum(h1, h2)), _s16(jnp.minimum(h1, h2))


_SC_MESH = plsc.VectorSubcoreMesh(core_axis_name="c", subcore_axis_name="s")


def _sc_b1(cp, cn):
  @functools.partial(
      pl.kernel, mesh=_SC_MESH,
      out_type=[
          jax.ShapeDtypeStruct((_NW, _C, _K), jnp.float32),
          jax.ShapeDtypeStruct((_NW, _C, _K), jnp.float32),
      ],
      compiler_params=pltpu.CompilerParams(needs_layout_passes=False),
      scratch_types=[
          pltpu.VMEM((_ROWS_PER_TILE, _C), jnp.float32),
          pltpu.VMEM((_C, _K), jnp.float32),
      ],
  )
  def b1(cp_hbm, cn_hbm, pp_out, pn_out, buf, ob):
    cid = lax.axis_index("c")
    sid = lax.axis_index("s")
    wid = cid * 16 + sid
    ngrp = _ROWS_PER_TILE // 16
    rows = [lax.iota(jnp.int32, 16) + 16 * g for g in range(ngrp)]
    neg16 = jnp.full((16,), _NEG, jnp.float32)

    def one_side(src_hbm, dst_hbm):
      pltpu.sync_copy(src_hbm.at[pl.ds(wid * _ROWS_PER_TILE, _ROWS_PER_TILE), :],
                      buf)

      def cls(c, _):
        cols = jnp.full((16,), 0, jnp.int32) + c
        v = [_s16(plsc.load_gather(buf, [rows[g], cols]))
             for g in range(ngrp)]
        hi, lo = v[0], neg16
        for g in range(1, ngrp):
          hi, lo = _merge32(hi, lo, v[g], neg16)
        ob[c, pl.ds(0, 16)] = hi
        ob[c, pl.ds(16, 16)] = lo
        return 0
      lax.fori_loop(0, _C, cls, 0)
      pltpu.sync_copy(ob, dst_hbm.at[wid])

    one_side(cp_hbm, pp_out)
    one_side(cn_hbm, pn_out)

  return b1(cp, cn)


def _sc_b2(pp, pn, mp, mn):
  cpt = _C // _NW

  @functools.partial(
      pl.kernel, mesh=_SC_MESH,
      out_type=[
          jax.ShapeDtypeStruct((_C, _K), jnp.float32),
          jax.ShapeDtypeStruct((_C, _K), jnp.float32),
          jax.ShapeDtypeStruct((_NW, 16), jnp.int32),
      ],
      compiler_params=pltpu.CompilerParams(needs_layout_passes=False),
      scratch_types=[
          pltpu.VMEM((_NW, _K), jnp.float32),
          pltpu.VMEM((_NB, _C), jnp.float32),
          pltpu.VMEM((cpt, _K), jnp.float32),
          pltpu.VMEM((16,), jnp.float32),
          pltpu.VMEM((16,), jnp.int32),
          pltpu.SemaphoreType.DMA,
      ],
  )
  def b2(pp_hbm, pn_hbm, mp_hbm, mn_hbm, dp_out, dn_out, tg_out,
         lists, mbuf, obuf, tbuf, trow, sem):
    cid = lax.axis_index("c")
    sid = lax.axis_index("s")
    wid = cid * 16 + sid
    c0 = wid * cpt
    rows16 = [lax.iota(jnp.int32, 16) + 16 * g for g in range(_NB // 16)]

    def side(src_hbm, m_hbm, dst_hbm, trig_in):
      pltpu.sync_copy(m_hbm, mbuf)

      def cls(j, trig):
        c = c0 + j
        for t in range(_NW):
          pltpu.async_copy(src_hbm.at[t, c], lists.at[t], sem)
        for t in range(_NW):
          pltpu.make_async_copy(src_hbm.at[t, c], lists.at[t], sem).wait()
        hi = lists[0, pl.ds(0, 16)]
        lo = lists[0, pl.ds(16, 16)]

        def mrg(t, carry):
          hi, lo = carry
          return _merge32(hi, lo, lists[t, pl.ds(0, 16)],
                          lists[t, pl.ds(16, 16)])
        hi, lo = lax.fori_loop(1, _NW, mrg, (hi, lo))
        obuf[j, pl.ds(0, 16)] = hi
        obuf[j, pl.ds(16, 16)] = lo
        tbuf[...] = lo
        t32 = plsc.load_gather(tbuf, [jnp.full((16,), 15, jnp.int32)])
        cols = jnp.full((16,), 0, jnp.int32) + c
        acc = jnp.full((16,), _NEG, jnp.float32)
        for g in range(_NB // 16):
          acc = jnp.maximum(acc, plsc.load_gather(mbuf, [rows16[g], cols]))
        hit = (acc >= t32) & (t32 > -1e29)
        nhit = lax.reduce_max(hit.astype(jnp.int32), (0,))
        live = (c < 100).astype(jnp.int32)
        return trig | (nhit * live)
      trig = lax.fori_loop(0, cpt, cls, trig_in)
      pltpu.sync_copy(obuf, dst_hbm.at[pl.ds(c0, cpt), :])
      return trig

    trig = side(pp_hbm, mp_hbm, dp_out, 0)
    trig = side(pn_hbm, mn_hbm, dn_out, trig)
    trow[...] = jnp.full((16,), 0, jnp.int32) + trig
    pltpu.sync_copy(trow, tg_out.at[wid])

  return b2(pp, pn, mp, mn)


def _tc_c_body(x_ref, y2_ref, y0_ref, dpt_ref, dnt_ref, tg_ref,
               ut_in, un_in, out_ref,
               bp_ref, bn_ref, dp_ref, dn_ref, trig_ref):
  b = pl.program_id(0)
  lanes = _lanes()

  @pl.when(b == 0)
  def _check():
    trig_ref[0] = (jnp.sum(tg_ref[...]) > 0).astype(jnp.int32)
    d_ap = jnp.sqrt(jnp.maximum(dpt_ref[...], 0.0))
    d_an = jnp.sqrt(jnp.maximum(-dnt_ref[...], 0.0))
    hinge = jnp.maximum(d_ap - d_an + _MARGIN, 0.0)
    lo = jnp.sum(hinge, axis=1, keepdims=True) / float(_K)
    present = dpt_ref[...][:, 0:1] > -1e29
    lo = jnp.where(present, lo, 0.0)
    n_present = jnp.sum(present.astype(jnp.float32))
    out_ref[...] = jnp.sum(lo, axis=0, keepdims=True) / n_present

  @pl.when(trig_ref[0] != 0)
  def _slow_block():
    x_blk = x_ref[...]
    cross = lax.dot_general(
        x_blk, ut_in[...], (((1,), (0,)), ((), ())),
        preferred_element_type=jnp.float32)
    xnorm = jnp.sum(x_blk * x_blk, axis=1, keepdims=True)
    d2 = jnp.maximum(un_in[...] - 2.0 * cross + xnorm, 0.0)
    ppos = jnp.where(y2_ref[...] == lanes, d2, _NEG)
    pneg = jnp.where(y0_ref[...] != lanes, -d2, _NEG)

    def extract(dst_ref, base, pv):
      def body(k, carry):
        pv, _ = carry
        m = jnp.max(pv, axis=0, keepdims=True)
        dst_ref[pl.ds(base + k, 1), :] = m
        return jnp.where(pv == m, _NEG, pv), m
      lax.fori_loop(0, _K, body,
                    (pv, jnp.full((1, _C), _NEG, jnp.float32)))
    extract(bp_ref, b * _K, ppos)
    extract(bn_ref, b * _K, pneg)

  @pl.when(jnp.logical_and(b == _NB - 1, trig_ref[0] != 0))
  def _slow_finish():
    def merge(src_ref, dst_ref):
      nchunks = (_NB * _K) // _CHUNK
      def kbody(k, _):
        def cmax(ci, m):
          blk = src_ref[pl.ds(ci * _CHUNK, _CHUNK), :]
          return jnp.maximum(m, jnp.max(blk, axis=0, keepdims=True))
        m = lax.fori_loop(0, nchunks, cmax,
                          jnp.full((1, _C), _NEG, jnp.float32))
        dst_ref[pl.ds(k, 1), :] = m
        def cupd(ci, _):
          blk = src_ref[pl.ds(ci * _CHUNK, _CHUNK), :]
          src_ref[pl.ds(ci * _CHUNK, _CHUNK), :] = jnp.where(
              blk == m, _NEG, blk)
          return 0
        lax.fori_loop(0, nchunks, cupd, 0)
        return 0
      lax.fori_loop(0, _K, kbody, 0)
    merge(bp_ref, dp_ref)
    merge(bn_ref, dn_ref)
    d_ap = jnp.sqrt(jnp.maximum(dp_ref[...], 0.0))
    d_an = jnp.sqrt(jnp.maximum(-dn_ref[...], 0.0))
    hinge = jnp.maximum(d_ap - d_an + _MARGIN, 0.0)
    lo = jnp.sum(hinge, axis=0, keepdims=True) / float(_K)
    present = dp_ref[...][0:1, :] > -1e29
    lo = jnp.where(present, lo, 0.0)
    n_present = jnp.sum(present.astype(jnp.float32))
    out_ref[...] = (jnp.sum(lo, axis=1, keepdims=True) / n_present).reshape(1, 1)


def _tc_c(x, y2, y0, dpt, dnt, tg, ut, un):
  return pl.pallas_call(
      _tc_c_body,
      grid=(_NB,),
      in_specs=[
          pl.BlockSpec((_B, _DIM), lambda b: (b, 0)),
          pl.BlockSpec((_B, 1), lambda b: (b, 0)),
          pl.BlockSpec((_B, 1), lambda b: (b, 0)),
          pl.BlockSpec((_C, _K), lambda b: (0, 0)),
          pl.BlockSpec((_C, _K), lambda b: (0, 0)),
          pl.BlockSpec((_NW, 16), lambda b: (0, 0)),
          pl.BlockSpec((_DIM, _C), lambda b: (0, 0)),
          pl.BlockSpec((1, _C), lambda b: (0, 0)),
      ],
      out_specs=pl.BlockSpec((1, 1), lambda b: (0, 0)),
      out_shape=jax.ShapeDtypeStruct((1, 1), jnp.float32),
      scratch_shapes=[
          pltpu.VMEM((_NB * _K, _C), jnp.float32),
          pltpu.VMEM((_NB * _K, _C), jnp.float32),
          pltpu.VMEM((_K, _C), jnp.float32),
          pltpu.VMEM((_K, _C), jnp.float32),
          pltpu.SMEM((1,), jnp.int32),
      ],
      compiler_params=pltpu.CompilerParams(
          dimension_semantics=("arbitrary",)),
  )(x, y2, y0, dpt, dnt, tg, ut, un)


@functools.partial(jax.jit, static_argnames=())
def kernel(x, y):
  y2 = y[:, 2:3]
  y0 = y[:, 0:1]
  cp, cn, mp, mn, ut, un = _tc_a(x, y2, y0)
  pp, pn = _sc_b1(cp, cn)
  dpt, dnt, tg = _sc_b2(pp, pn, mp, mn)
  out = _tc_c(x, y2, y0, dpt, dnt, tg, ut, un)
  return out.reshape((1,))
